# baseline (speedup 1.0000x reference)
import functools

import jax
import jax.numpy as jnp
from jax.experimental import pallas as pl
from jax.experimental.pallas import tpu as pltpu


def _decoder_kernel(seq_len, B, N, HID, st_ref, h0i_ref, w1gh_ref, w1gi_ref,
                    b1g_ref, w1ch_ref, w1ci_ref, b1c_ref, w2gh_ref, w2gi_ref,
                    b2g_ref, w2ch_ref, w2ci_ref, b2c_ref, wpt_ref, bp_ref,
                    out_ref):
    def matmul(a, b):
        return jax.lax.dot(a, b, preferred_element_type=jnp.float32)

    def diffuse(x):
        return jnp.concatenate(
            [matmul(x[:, b * N:(b + 1) * N], st_ref[...]) for b in range(B)],
            axis=1)

    def cell(inp, h, wgh_ref, wgi_ref, bg_ref, wch_ref, wci_ref, bc_ref):
        y0 = jnp.concatenate([h, inp], axis=0)
        y1 = diffuse(y0)
        y2 = 2.0 * diffuse(y1) - y0
        g = bg_ref[...]
        for k, yk in enumerate((y0, y1, y2)):
            g = (g + matmul(wgh_ref[k], yk[:HID])
                 + matmul(wgi_ref[k], yk[HID:]))
        g = jax.nn.sigmoid(g)
        r, u = g[:HID], g[HID:]
        rh0 = r * h
        rh1 = diffuse(rh0)
        rh2 = 2.0 * diffuse(rh1) - rh0
        c = bc_ref[...]
        for k, (rhk, yk) in enumerate(((rh0, y0), (rh1, y1), (rh2, y2))):
            c = c + matmul(wch_ref[k], rhk) + matmul(wci_ref[k], yk[HID:])
        c = jnp.tanh(c)
        return u * h + (1.0 - u) * c

    h0 = h0i_ref[0]
    h1 = h0i_ref[1]
    cur = jnp.zeros((8, B * N), jnp.float32)
    for t in range(seq_len):
        h0 = cell(cur, h0, w1gh_ref, w1gi_ref, b1g_ref,
                  w1ch_ref, w1ci_ref, b1c_ref)
        h1 = cell(h0, h1, w2gh_ref, w2gi_ref, b2g_ref,
                  w2ch_ref, w2ci_ref, b2c_ref)
        proj = matmul(wpt_ref[...], h1) + bp_ref[...]
        out_ref[t] = proj
        cur = jnp.concatenate(
            [proj, jnp.zeros((7, B * N), jnp.float32)], axis=0)


def kernel(inputs, initial_hidden_state, supports, W1_gate, b1_gate,
           W1_cand, b1_cand, W2_gate, b2_gate, W2_cand, b2_cand, Wp, bp):
    seq_len, B = inputs.shape[0], inputs.shape[1]
    N = supports.shape[1]
    HID = Wp.shape[0]
    OUT_DIM = Wp.shape[1]
    num_layers = initial_hidden_state.shape[0]
    nm = 3

    St = supports[0].T
    h0i = (initial_hidden_state.reshape(num_layers, B, N, HID)
           .transpose(0, 3, 1, 2).reshape(num_layers, HID, B * N))

    w1g = W1_gate.reshape(1 + HID, nm, 2 * HID)
    w1c = W1_cand.reshape(1 + HID, nm, HID)
    pad = jnp.zeros((7, nm, 2 * HID), jnp.float32)
    padc = jnp.zeros((7, nm, HID), jnp.float32)
    w1gh = w1g[1:].transpose(1, 2, 0)
    w1gi = jnp.concatenate([w1g[:1], pad], 0).transpose(1, 2, 0)
    w1ch = w1c[1:].transpose(1, 2, 0)
    w1ci = jnp.concatenate([w1c[:1], padc], 0).transpose(1, 2, 0)
    w2g = W2_gate.reshape(2 * HID, nm, 2 * HID)
    w2gi = w2g[:HID].transpose(1, 2, 0)
    w2gh = w2g[HID:].transpose(1, 2, 0)
    w2c = W2_cand.reshape(2 * HID, nm, HID)
    w2ci = w2c[:HID].transpose(1, 2, 0)
    w2ch = w2c[HID:].transpose(1, 2, 0)

    b1g = b1_gate.reshape(2 * HID, 1)
    b1c = b1_cand.reshape(HID, 1)
    b2g = b2_gate.reshape(2 * HID, 1)
    b2c = b2_cand.reshape(HID, 1)
    wpt = Wp.T
    bp2 = bp.reshape(1, 1)

    body = functools.partial(_decoder_kernel, seq_len, B, N, HID)
    out = pl.pallas_call(
        body,
        out_shape=jax.ShapeDtypeStruct((seq_len, 1, B * N), jnp.float32),
    )(St, h0i, w1gh, w1gi, b1g, w1ch, w1ci, b1c, w2gh, w2gi, b2g,
      w2ch, w2ci, b2c, wpt, bp2)

    return out.reshape(seq_len, B, N * OUT_DIM)
